# baseline (speedup 1.0000x reference)
import functools

import jax
import jax.numpy as jnp
import numpy as np
from jax.experimental import pallas as pl
from jax.experimental.pallas import tpu as pltpu

_NORBS = [1, 1, 3, 3, 5]
_FULL = 13
_FEAT = 107
_N_ATOMS = 512
_N_EDGES = 8192
_BM = 2048
_M_PAD = 2 * _N_EDGES + _BM
_ONS_BASE = 2 * _N_EDGES
_GA = 8
_N_STRIPES = _N_ATOMS // _GA
_N_UPD = 4 * _N_EDGES


def _expansion_matrices():
    starts = np.cumsum([0] + _NORBS)[:-1]
    shell_of = np.zeros(_FULL, np.int32)
    local_of = np.zeros(_FULL, np.int32)
    for s, (st, n) in enumerate(zip(starts, _NORBS)):
        shell_of[st:st + n] = s
        local_of[st:st + n] = np.arange(n)
    off = {}
    o = 0
    for i, ni in enumerate(_NORBS):
        for j, nj in enumerate(_NORBS):
            if i <= j:
                off[(i, j)] = o
                o += ni * nj
    G = np.zeros((_FEAT, _FULL * _FULL), np.float32)
    for r in range(_FULL):
        for c in range(_FULL):
            i, j = shell_of[r], shell_of[c]
            if i <= j:
                f = off[(i, j)] + local_of[r] * _NORBS[j] + local_of[c]
                G[f, r * _FULL + c] = 0.5 if i == j else 1.0
    GT = np.zeros_like(G)
    for r in range(_FULL):
        for c in range(_FULL):
            GT[:, r * _FULL + c] = G[:, c * _FULL + r]
    GS = G + GT
    W = np.zeros((_FULL, 3, _FEAT, 16), np.float32)
    for i in range(_FULL):
        W[i, 0, :, :13] = G[:, i * _FULL:(i + 1) * _FULL]
        W[i, 1, :, :13] = GT[:, i * _FULL:(i + 1) * _FULL]
        W[i, 2, :, :13] = GS[:, i * _FULL:(i + 1) * _FULL]
    return W


_W = _expansion_matrices()


def _expand_body(f_ref, w_ref, y_ref):
    f = f_ref[...]
    for i in range(_FULL):
        y_ref[i] = jnp.dot(f, w_ref[i, 0],
                           preferred_element_type=jnp.float32)


_ACC_W = 6784


def _scatter_body(u_ref, offs_ref, ids_ref, out_ref, *accs):
    g = pl.program_id(0)
    for a in accs:
        a[...] = jnp.zeros((16 * _GA, _ACC_W), jnp.float32)
    lane = jax.lax.broadcasted_iota(jnp.int32, (16, 256), 1)
    zrows = jnp.zeros((3, 256), jnp.float32)

    def one(p, acc):
        base_u = pl.multiple_of((p & 0xFFF) * 128, 128)
        off_u = ((p >> 12) & 7) * 16
        base_a = pl.multiple_of(((p >> 15) & 0x3F) * 128, 128)
        off_a = (p >> 21) & 0x7F
        base_r = pl.multiple_of(((p >> 28) & 7) * 16, 16)
        w = jnp.concatenate([u_ref[:, pl.ds(base_u, 256)], zrows], axis=0)
        w = pltpu.roll(w, (off_a - off_u) & 255, axis=1)
        w = jnp.where((lane >= off_a) & (lane < off_a + 13), w, 0.0)
        acc[pl.ds(base_r, 16), pl.ds(base_a, 256)] = (
            acc[pl.ds(base_r, 16), pl.ds(base_a, 256)] + w)

    start = offs_ref[g]
    nu = len(accs)

    def body(t, carry):
        b = start + nu * t
        for q, a in enumerate(accs):
            one(ids_ref[b + q], a)
        return carry

    jax.lax.fori_loop(0, (offs_ref[g + 1] - start) // nu, body, 0)
    n = _N_ATOMS * _FULL
    for a_i in range(_GA):
        tot = accs[0][pl.ds(16 * a_i, 13), :n]
        for a in accs[1:]:
            tot = tot + a[pl.ds(16 * a_i, 13), :n]
        out_ref[0, pl.ds(13 * a_i, 13), :] = tot


def kernel(edge_features, node_features, atom_types, edge_index):
    del atom_types
    E = _N_EDGES
    F = jnp.concatenate(
        [edge_features, edge_features,
         jnp.pad(node_features, ((0, _BM - _N_ATOMS), (0, 0)))], axis=0)
    n_mb = _M_PAD // _BM
    e_mb = E // _BM
    Wc = jnp.asarray(_W)
    Y = pl.pallas_call(
        _expand_body,
        grid=(n_mb,),
        in_specs=[
            pl.BlockSpec((_BM, _FEAT), lambda m: (m, 0)),
            pl.BlockSpec((_FULL, 1, _FEAT, 16),
                         lambda m: (0, (m >= e_mb).astype(jnp.int32)
                                    + (m >= 2 * e_mb).astype(jnp.int32),
                                    0, 0)),
        ],
        out_specs=pl.BlockSpec((_FULL, _BM, 16), lambda m: (0, m, 0)),
        out_shape=jax.ShapeDtypeStruct((_FULL, _M_PAD, 16), jnp.float32),
    )(F, Wc)
    U13 = Y.reshape(_FULL, _M_PAD * 16)

    src = edge_index[0].astype(jnp.int32)
    dst = edge_index[1].astype(jnp.int32)
    atoms = jnp.arange(_N_ATOMS, dtype=jnp.int32)
    row = jnp.concatenate([src, dst, atoms])
    colv = jnp.concatenate([dst, src, atoms])
    uid = jnp.concatenate([jnp.arange(2 * E, dtype=jnp.int32),
                           _ONS_BASE + atoms])
    ca = colv * 13
    packed = ((uid // 8) | ((uid % 8) << 12)
              | ((ca // 128) << 15) | ((ca % 128) << 21)
              | ((row & (_GA - 1)) << 28))
    perm = jnp.argsort(row)
    row_s = row[perm]
    band_s = row_s // _GA
    ids = packed[perm]
    offs = jnp.searchsorted(
        band_s, jnp.arange(_N_STRIPES + 1, dtype=jnp.int32),
        side='left').astype(jnp.int32)
    counts = offs[1:] - offs[:-1]
    offs2 = jnp.concatenate([
        jnp.zeros((1,), jnp.int32),
        jnp.cumsum((counts + 3) // 4 * 4, dtype=jnp.int32)])
    n_ids2 = ids.shape[0] + 3 * _N_STRIPES
    dummy = jnp.int32(17000 // 8)
    pos = offs2[band_s] + (jnp.arange(ids.shape[0], dtype=jnp.int32)
                           - offs[band_s])
    ids2 = jnp.full((n_ids2,), dummy, jnp.int32).at[pos].set(ids)

    out = pl.pallas_call(
        _scatter_body,
        grid=(_N_STRIPES,),
        in_specs=[
            pl.BlockSpec((_FULL, _M_PAD * 16), lambda g: (0, 0)),
            pl.BlockSpec(memory_space=pltpu.SMEM),
            pl.BlockSpec(memory_space=pltpu.SMEM),
        ],
        out_specs=pl.BlockSpec((1, _GA * _FULL, _N_ATOMS * _FULL),
                               lambda g: (g, 0, 0)),
        out_shape=jax.ShapeDtypeStruct(
            (_N_STRIPES, _GA * _FULL, _N_ATOMS * _FULL), jnp.float32),
        scratch_shapes=[pltpu.VMEM((16 * _GA, _ACC_W), jnp.float32)
                        for _ in range(4)],
    )(U13, offs2, ids2)
    return out.reshape(_N_ATOMS * _FULL, _N_ATOMS * _FULL)


# speedup vs baseline: 1.3303x; 1.3303x over previous
import functools

import jax
import jax.numpy as jnp
import numpy as np
from jax import lax
from jax.experimental import pallas as pl
from jax.experimental.pallas import tpu as pltpu
from jax.experimental.pallas import tpu_sc as plsc

_NORBS = [1, 1, 3, 3, 5]
_FULL = 13
_FEAT = 107
_N_ATOMS = 512
_N_EDGES = 8192
_BM = 2048
_M_PAD = 2 * _N_EDGES + _BM
_ONS_BASE = 2 * _N_EDGES
_GA = 8
_N_STRIPES = _N_ATOMS // _GA
_N_UPD = 4 * _N_EDGES


def _expansion_matrices():
    starts = np.cumsum([0] + _NORBS)[:-1]
    shell_of = np.zeros(_FULL, np.int32)
    local_of = np.zeros(_FULL, np.int32)
    for s, (st, n) in enumerate(zip(starts, _NORBS)):
        shell_of[st:st + n] = s
        local_of[st:st + n] = np.arange(n)
    off = {}
    o = 0
    for i, ni in enumerate(_NORBS):
        for j, nj in enumerate(_NORBS):
            if i <= j:
                off[(i, j)] = o
                o += ni * nj
    G = np.zeros((_FEAT, _FULL * _FULL), np.float32)
    for r in range(_FULL):
        for c in range(_FULL):
            i, j = shell_of[r], shell_of[c]
            if i <= j:
                f = off[(i, j)] + local_of[r] * _NORBS[j] + local_of[c]
                G[f, r * _FULL + c] = 0.5 if i == j else 1.0
    GT = np.zeros_like(G)
    for r in range(_FULL):
        for c in range(_FULL):
            GT[:, r * _FULL + c] = G[:, c * _FULL + r]
    GS = G + GT
    W = np.zeros((_FULL, 3, _FEAT, 16), np.float32)
    for i in range(_FULL):
        W[i, 0, :, :13] = G[:, i * _FULL:(i + 1) * _FULL]
        W[i, 1, :, :13] = GT[:, i * _FULL:(i + 1) * _FULL]
        W[i, 2, :, :13] = GS[:, i * _FULL:(i + 1) * _FULL]
    return W


_W = _expansion_matrices()


def _expand_body(f_ref, w_ref, y_ref):
    f = f_ref[...]
    for i in range(_FULL):
        y_ref[i] = jnp.dot(f, w_ref[i, 0],
                           preferred_element_type=jnp.float32)


_ACC_W = 6784
_CAP = 18432
_DUMMY = 17000 // 8
_SEG = 1040


def _route_sc(edge_index):
    E = _N_EDGES
    mesh = plsc.VectorSubcoreMesh(core_axis_name="c", subcore_axis_name="s")

    @functools.partial(
        pl.kernel, mesh=mesh,
        compiler_params=pltpu.CompilerParams(needs_layout_passes=False),
        out_type=(jax.ShapeDtypeStruct((_N_STRIPES * _CAP,), jnp.int32),
                  jax.ShapeDtypeStruct((1024,), jnp.int32)),
        scratch_types=[pltpu.VMEM((E,), jnp.int32),
                       pltpu.VMEM((E,), jnp.int32),
                       pltpu.VMEM((_CAP,), jnp.int32),
                       pltpu.VMEM((_CAP,), jnp.int32)],
    )
    def k(ei, regions, cnts, srcb, dstb, b0, b1):
        w = lax.axis_index("s") * 2 + lax.axis_index("c")
        pltpu.sync_copy(ei.at[0], srcb)
        pltpu.sync_copy(ei.at[1], dstb)
        iota = lax.iota(jnp.int32, 16)
        lo = w * 16

        def pack(uidv, colv, rowv):
            cav = colv * 13
            return ((uidv >> 3) | ((uidv & 7) << 12)
                    | ((cav >> 7) << 15) | ((cav & 127) << 21)
                    | ((rowv & 7) << 28))

        def append(buf, cb, m, vals):
            plsc.store_scatter(buf, [iota * _SEG + cb], vals, mask=m)
            return cb + m.astype(jnp.int32)

        def passa(i, carry):
            c0, c1 = carry
            s = srcb[pl.ds(i * 16, 16)]
            d = dstb[pl.ds(i * 16, 16)]
            u = i * 16 + iota
            for rowv, colv, uidv in ((s, d, u), (d, s, u + E)):
                pk = pack(uidv, colv, rowv)
                bandv = rowv >> 3
                c0 = append(b0, c0, bandv == 2 * w, pk)
                c1 = append(b1, c1, bandv == 2 * w + 1, pk)
            return c0, c1

        z16 = jnp.zeros((16,), jnp.int32)
        c0, c1 = lax.fori_loop(0, E // 16, passa, (z16, z16))
        rowv = lo + iota
        pk = pack(rowv + 2 * E, rowv, rowv)
        bandv = rowv >> 3
        c0 = append(b0, c0, bandv == 2 * w, pk)
        c1 = append(b1, c1, bandv == 2 * w + 1, pk)

        dum = jnp.full((16,), _DUMMY, jnp.int32)
        for b, (buf, cb) in enumerate(((b0, c0), (b1, c1))):
            band = 2 * w + b
            for j in range(3):
                plsc.store_scatter(buf, [iota * _SEG + cb + j], dum)
            cv = (cb + 3) // 4 * 4
            pltpu.sync_copy(
                buf,
                regions.at[pl.ds(pl.multiple_of(band * _CAP, 1024), _CAP)])
            buf[pl.ds(0, 16)] = cv
            pltpu.sync_copy(
                buf.at[pl.ds(0, 16)],
                cnts.at[pl.ds(pl.multiple_of(band * 16, 16), 16)])

    return k(edge_index)


def _scatter_body(u_ref, cnts_ref, ids_ref, out_ref, *accs):
    g = pl.program_id(0)
    for a in accs:
        a[...] = jnp.zeros((16 * _GA, _ACC_W), jnp.float32)
    lane = jax.lax.broadcasted_iota(jnp.int32, (16, 256), 1)
    zrows = jnp.zeros((3, 256), jnp.float32)

    def one(p, acc):
        base_u = pl.multiple_of((p & 0xFFF) * 128, 128)
        off_u = ((p >> 12) & 7) * 16
        base_a = pl.multiple_of(((p >> 15) & 0x3F) * 128, 128)
        off_a = (p >> 21) & 0x7F
        base_r = pl.multiple_of(((p >> 28) & 7) * 16, 16)
        w = jnp.concatenate([u_ref[:, pl.ds(base_u, 256)], zrows], axis=0)
        w = pltpu.roll(w, (off_a - off_u) & 255, axis=1)
        w = jnp.where((lane >= off_a) & (lane < off_a + 13), w, 0.0)
        acc[pl.ds(base_r, 16), pl.ds(base_a, 256)] = (
            acc[pl.ds(base_r, 16), pl.ds(base_a, 256)] + w)

    nu = len(accs)
    for seg in range(16):
        sbase = seg * _SEG

        def body(t, carry):
            b = sbase + nu * t
            for q, a in enumerate(accs):
                one(ids_ref[0, 0, b + q], a)
            return carry

        jax.lax.fori_loop(0, cnts_ref[g, seg] // nu, body, 0)
    n = _N_ATOMS * _FULL
    for a_i in range(_GA):
        tot = accs[0][pl.ds(16 * a_i, 13), :n]
        for a in accs[1:]:
            tot = tot + a[pl.ds(16 * a_i, 13), :n]
        out_ref[0, pl.ds(13 * a_i, 13), :] = tot


def kernel(edge_features, node_features, atom_types, edge_index):
    del atom_types
    E = _N_EDGES
    F = jnp.concatenate(
        [edge_features, edge_features,
         jnp.pad(node_features, ((0, _BM - _N_ATOMS), (0, 0)))], axis=0)
    n_mb = _M_PAD // _BM
    e_mb = E // _BM
    Wc = jnp.asarray(_W)
    Y = pl.pallas_call(
        _expand_body,
        grid=(n_mb,),
        in_specs=[
            pl.BlockSpec((_BM, _FEAT), lambda m: (m, 0)),
            pl.BlockSpec((_FULL, 1, _FEAT, 16),
                         lambda m: (0, (m >= e_mb).astype(jnp.int32)
                                    + (m >= 2 * e_mb).astype(jnp.int32),
                                    0, 0)),
        ],
        out_specs=pl.BlockSpec((_FULL, _BM, 16), lambda m: (0, m, 0)),
        out_shape=jax.ShapeDtypeStruct((_FULL, _M_PAD, 16), jnp.float32),
    )(F, Wc)
    U13 = Y.reshape(_FULL, _M_PAD * 16)

    regions, cnts = _route_sc(edge_index.astype(jnp.int32))

    out = pl.pallas_call(
        _scatter_body,
        grid=(_N_STRIPES,),
        in_specs=[
            pl.BlockSpec((_FULL, _M_PAD * 16), lambda g: (0, 0)),
            pl.BlockSpec(memory_space=pltpu.SMEM),
            pl.BlockSpec((1, 1, _CAP), lambda g: (g, 0, 0),
                         memory_space=pltpu.SMEM),
        ],
        out_specs=pl.BlockSpec((1, _GA * _FULL, _N_ATOMS * _FULL),
                               lambda g: (g, 0, 0)),
        out_shape=jax.ShapeDtypeStruct(
            (_N_STRIPES, _GA * _FULL, _N_ATOMS * _FULL), jnp.float32),
        scratch_shapes=[pltpu.VMEM((16 * _GA, _ACC_W), jnp.float32)
                        for _ in range(4)],
    )(U13, cnts.reshape(_N_STRIPES, 16), regions.reshape(_N_STRIPES, 1, _CAP))
    return out.reshape(_N_ATOMS * _FULL, _N_ATOMS * _FULL)


# speedup vs baseline: 1.3305x; 1.0001x over previous
import functools

import jax
import jax.numpy as jnp
import numpy as np
from jax import lax
from jax.experimental import pallas as pl
from jax.experimental.pallas import tpu as pltpu
from jax.experimental.pallas import tpu_sc as plsc

_NORBS = [1, 1, 3, 3, 5]
_FULL = 13
_FEAT = 107
_N_ATOMS = 512
_N_EDGES = 8192
_BM = 2048
_M_PAD = 2 * _N_EDGES + _BM
_ONS_BASE = 2 * _N_EDGES
_GA = 8
_N_STRIPES = _N_ATOMS // _GA


def _expansion_matrices():
    starts = np.cumsum([0] + _NORBS)[:-1]
    shell_of = np.zeros(_FULL, np.int32)
    local_of = np.zeros(_FULL, np.int32)
    for s, (st, n) in enumerate(zip(starts, _NORBS)):
        shell_of[st:st + n] = s
        local_of[st:st + n] = np.arange(n)
    off = {}
    o = 0
    for i, ni in enumerate(_NORBS):
        for j, nj in enumerate(_NORBS):
            if i <= j:
                off[(i, j)] = o
                o += ni * nj
    G = np.zeros((_FEAT, _FULL * _FULL), np.float32)
    for r in range(_FULL):
        for c in range(_FULL):
            i, j = shell_of[r], shell_of[c]
            if i <= j:
                f = off[(i, j)] + local_of[r] * _NORBS[j] + local_of[c]
                G[f, r * _FULL + c] = 0.5 if i == j else 1.0
    GT = np.zeros_like(G)
    for r in range(_FULL):
        for c in range(_FULL):
            GT[:, r * _FULL + c] = G[:, c * _FULL + r]
    GS = G + GT
    W = np.zeros((_FULL, 3, _FEAT, 16), np.float32)
    for i in range(_FULL):
        W[i, 0, :, :13] = G[:, i * _FULL:(i + 1) * _FULL]
        W[i, 1, :, :13] = GT[:, i * _FULL:(i + 1) * _FULL]
        W[i, 2, :, :13] = GS[:, i * _FULL:(i + 1) * _FULL]
    return W


_W = _expansion_matrices()


def _expand_body(f_ref, w_ref, y_ref):
    f = f_ref[...]
    for i in range(_FULL):
        y_ref[i] = jnp.dot(f, w_ref[i, 0],
                           preferred_element_type=jnp.float32)


_ACC_W = 6784
_CAP = 18432
_DUMMY = 17000 // 8
_SEG = 1040


def _route_sc(edge_index):
    E = _N_EDGES
    mesh = plsc.VectorSubcoreMesh(core_axis_name="c", subcore_axis_name="s")

    @functools.partial(
        pl.kernel, mesh=mesh,
        compiler_params=pltpu.CompilerParams(needs_layout_passes=False),
        out_type=(jax.ShapeDtypeStruct((_N_STRIPES * _CAP,), jnp.int32),
                  jax.ShapeDtypeStruct((1024,), jnp.int32)),
        scratch_types=[pltpu.VMEM((E,), jnp.int32),
                       pltpu.VMEM((E,), jnp.int32),
                       pltpu.VMEM((_CAP,), jnp.int32),
                       pltpu.VMEM((_CAP,), jnp.int32)],
    )
    def k(ei, regions, cnts, srcb, dstb, b0, b1):
        w = lax.axis_index("s") * 2 + lax.axis_index("c")
        pltpu.sync_copy(ei.at[0], srcb)
        pltpu.sync_copy(ei.at[1], dstb)
        iota = lax.iota(jnp.int32, 16)
        lo = w * 16

        def pack(uidv, colv, rowv):
            cav = colv * 13
            return ((uidv >> 3) | ((uidv & 7) << 12)
                    | ((cav >> 7) << 15) | ((cav & 127) << 21)
                    | ((rowv & 7) << 28))

        def append(buf, cb, m, vals):
            plsc.store_scatter(buf, [iota * _SEG + cb], vals, mask=m)
            return cb + m.astype(jnp.int32)

        def passa(i, carry):
            c0, c1 = carry
            s = srcb[pl.ds(i * 16, 16)]
            d = dstb[pl.ds(i * 16, 16)]
            u = i * 16 + iota
            for rowv, colv, uidv in ((s, d, u), (d, s, u + E)):
                pk = pack(uidv, colv, rowv)
                bandv = rowv >> 3
                c0 = append(b0, c0, bandv == 2 * w, pk)
                c1 = append(b1, c1, bandv == 2 * w + 1, pk)
            return c0, c1

        z16 = jnp.zeros((16,), jnp.int32)
        c0, c1 = lax.fori_loop(0, E // 16, passa, (z16, z16))
        rowv = lo + iota
        pk = pack(rowv + 2 * E, rowv, rowv)
        bandv = rowv >> 3
        c0 = append(b0, c0, bandv == 2 * w, pk)
        c1 = append(b1, c1, bandv == 2 * w + 1, pk)

        dum = jnp.full((16,), _DUMMY, jnp.int32)
        for b, (buf, cb) in enumerate(((b0, c0), (b1, c1))):
            band = 2 * w + b
            for j in range(3):
                plsc.store_scatter(buf, [iota * _SEG + cb + j], dum)
            cv = (cb + 3) // 4 * 4
            pltpu.sync_copy(
                buf,
                regions.at[pl.ds(pl.multiple_of(band * _CAP, 1024), _CAP)])
            buf[pl.ds(0, 16)] = cv
            pltpu.sync_copy(
                buf.at[pl.ds(0, 16)],
                cnts.at[pl.ds(pl.multiple_of(band * 16, 16), 16)])

    return k(edge_index)


def _scatter_body(u_ref, cnts_ref, ids_ref, out_ref, *accs):
    g = pl.program_id(0)
    for a in accs:
        a[...] = jnp.zeros((16 * _GA, _ACC_W), jnp.float32)
    lane = jax.lax.broadcasted_iota(jnp.int32, (16, 256), 1)
    zrows = jnp.zeros((3, 256), jnp.float32)

    def one(p, acc):
        base_u = pl.multiple_of((p & 0xFFF) * 128, 128)
        off_u = ((p >> 12) & 7) * 16
        base_a = pl.multiple_of(((p >> 15) & 0x3F) * 128, 128)
        off_a = (p >> 21) & 0x7F
        base_r = pl.multiple_of(((p >> 28) & 7) * 16, 16)
        w = jnp.concatenate([u_ref[:, pl.ds(base_u, 256)], zrows], axis=0)
        w = pltpu.roll(w, (off_a - off_u) & 255, axis=1)
        w = jnp.where((lane >= off_a) & (lane < off_a + 13), w, 0.0)
        acc[pl.ds(base_r, 16), pl.ds(base_a, 256)] = (
            acc[pl.ds(base_r, 16), pl.ds(base_a, 256)] + w)

    nu = len(accs)
    for seg in range(16):
        sbase = seg * _SEG

        def body(t, carry):
            b = sbase + nu * t
            for q, a in enumerate(accs):
                one(ids_ref[0, 0, b + q], a)
            return carry

        jax.lax.fori_loop(0, cnts_ref[g, seg] // nu, body, 0)
    n = _N_ATOMS * _FULL
    for a_i in range(_GA):
        tot = accs[0][pl.ds(16 * a_i, 13), :n]
        for a in accs[1:]:
            tot = tot + a[pl.ds(16 * a_i, 13), :n]
        out_ref[0, pl.ds(13 * a_i, 13), :] = tot


def kernel(edge_features, node_features, atom_types, edge_index):
    del atom_types
    E = _N_EDGES
    F = jnp.concatenate(
        [edge_features, edge_features,
         jnp.pad(node_features, ((0, _BM - _N_ATOMS), (0, 0)))], axis=0)
    n_mb = _M_PAD // _BM
    e_mb = E // _BM
    Wc = jnp.asarray(_W)
    Y = pl.pallas_call(
        _expand_body,
        grid=(n_mb,),
        in_specs=[
            pl.BlockSpec((_BM, _FEAT), lambda m: (m, 0)),
            pl.BlockSpec((_FULL, 1, _FEAT, 16),
                         lambda m: (0, (m >= e_mb).astype(jnp.int32)
                                    + (m >= 2 * e_mb).astype(jnp.int32),
                                    0, 0)),
        ],
        out_specs=pl.BlockSpec((_FULL, _BM, 16), lambda m: (0, m, 0)),
        out_shape=jax.ShapeDtypeStruct((_FULL, _M_PAD, 16), jnp.float32),
    )(F, Wc)
    U13 = Y.reshape(_FULL, _M_PAD * 16)

    regions, cnts = _route_sc(edge_index.astype(jnp.int32))

    out = pl.pallas_call(
        _scatter_body,
        grid=(_N_STRIPES,),
        in_specs=[
            pl.BlockSpec((_FULL, _M_PAD * 16), lambda g: (0, 0)),
            pl.BlockSpec(memory_space=pltpu.SMEM),
            pl.BlockSpec((1, 1, _CAP), lambda g: (g, 0, 0),
                         memory_space=pltpu.SMEM),
        ],
        out_specs=pl.BlockSpec((1, _GA * _FULL, _N_ATOMS * _FULL),
                               lambda g: (g, 0, 0)),
        out_shape=jax.ShapeDtypeStruct(
            (_N_STRIPES, _GA * _FULL, _N_ATOMS * _FULL), jnp.float32),
        scratch_shapes=[pltpu.VMEM((16 * _GA, _ACC_W), jnp.float32)
                        for _ in range(4)],
    )(U13, cnts.reshape(_N_STRIPES, 16), regions.reshape(_N_STRIPES, 1, _CAP))
    return out.reshape(_N_ATOMS * _FULL, _N_ATOMS * _FULL)
